# baseline (speedup 1.0000x reference)
import jax
import jax.numpy as jnp
from jax import lax
from jax.experimental import pallas as pl
from jax.experimental.pallas import tpu as pltpu
from jax.experimental.pallas import tpu_sc as plsc

EMBED = 768
INTER = 1024
NEXP = 64
NTOK = 128

_INFO = plsc.get_sparse_core_info()
_NC = 2
_NS = _INFO.num_subcores
_L = _INFO.num_lanes
_NW = _NC * _NS
_TPW = NTOK // _NW
_CH = NEXP // _L


def _gate_body(x_ref, wg_ref, g_ref, xb_ref):
    x = x_ref[...].reshape(NTOK, EMBED)
    g_ref[...] = jnp.dot(x, wg_ref[...], preferred_element_type=jnp.float32)
    xb_ref[...] = x.astype(jnp.bfloat16)


def _route_body(g_hbm, ew_hbm, g_v, ew_v, bf1, bf2, bi1, bi2, sem):
    wid = lax.axis_index("s") * _NC + lax.axis_index("c")
    base = wid * _TPW
    pltpu.sync_copy(g_hbm.at[pl.ds(base, _TPW), :], g_v)
    iota = lax.iota(jnp.int32, _L)

    def _rot(x, s, buf):
        buf[pl.ds(0, _L)] = x
        buf[pl.ds(_L, _L)] = x
        return buf[pl.ds(s, _L)]

    def _token(k, _):
        v1 = g_v[k, pl.ds(0, _L)]
        i1 = iota
        v2 = jnp.full((_L,), -jnp.inf, jnp.float32)
        i2 = jnp.full((_L,), NEXP, jnp.int32)
        for c in range(1, _CH):
            g = g_v[k, pl.ds(c * _L, _L)]
            gi = iota + c * _L
            is1 = g > v1
            gt2 = g > v2
            v2 = jnp.where(is1, v1, jnp.where(gt2, g, v2))
            i2 = jnp.where(is1, i1, jnp.where(gt2, gi, i2))
            v1 = jnp.where(is1, g, v1)
            i1 = jnp.where(is1, gi, i1)
        for s in (8, 4, 2, 1):
            rv1 = _rot(v1, s, bf1)
            ri1 = _rot(i1, s, bi1)
            rv2 = _rot(v2, s, bf2)
            ri2 = _rot(i2, s, bi2)
            a_top = (v1 > rv1) | ((v1 == rv1) & (i1 < ri1))
            n1 = jnp.where(a_top, v1, rv1)
            j1 = jnp.where(a_top, i1, ri1)
            lo_v = jnp.where(a_top, rv1, v1)
            lo_i = jnp.where(a_top, ri1, i1)
            cd_v = jnp.where(a_top, v2, rv2)
            cd_i = jnp.where(a_top, i2, ri2)
            lo_top = (lo_v > cd_v) | ((lo_v == cd_v) & (lo_i < cd_i))
            v2 = jnp.where(lo_top, lo_v, cd_v)
            i2 = jnp.where(lo_top, lo_i, cd_i)
            v1, i1 = n1, j1
        w1v = 1.0 / (1.0 + jnp.exp(v2 - v1))
        w2v = 1.0 - w1v
        zero = jnp.zeros((_L,), jnp.float32)
        for c in range(_CH):
            ic = iota + c * _L
            val = (jnp.where(ic == i1, w1v, zero)
                   + jnp.where(ic == i2, w2v, zero))
            ew_v[k, pl.ds(c * _L, _L)] = val
        return 0

    lax.fori_loop(0, _TPW, _token, 0)
    pltpu.sync_copy(ew_v, ew_hbm.at[pl.ds(base, _TPW), :])


def _routing_sc(g):
    mesh = plsc.VectorSubcoreMesh(core_axis_name="c", subcore_axis_name="s",
                                  num_cores=_NC)
    return pl.kernel(
        _route_body,
        out_type=jax.ShapeDtypeStruct((NTOK, NEXP), jnp.float32),
        mesh=mesh,
        scratch_types=[
            pltpu.VMEM((_TPW, NEXP), jnp.float32),
            pltpu.VMEM((_TPW, NEXP), jnp.float32),
            pltpu.VMEM((2 * _L,), jnp.float32),
            pltpu.VMEM((2 * _L,), jnp.float32),
            pltpu.VMEM((2 * _L,), jnp.int32),
            pltpu.VMEM((2 * _L,), jnp.int32),
            pltpu.SemaphoreType.DMA,
        ],
    )(g)


def _ffn_body(ew_ref, xb_ref, w1_ref, w2_ref, w3_ref, out_ref, acc_ref):
    e = pl.program_id(0)

    @pl.when(e == 0)
    def _():
        acc_ref[...] = jnp.zeros_like(acc_ref)

    xb = xb_ref[...]
    w1 = w1_ref[0].astype(jnp.bfloat16)
    w3 = w3_ref[0].astype(jnp.bfloat16)
    w2 = w2_ref[0].astype(jnp.bfloat16)
    h1 = jnp.dot(xb, w1, preferred_element_type=jnp.float32)
    h3 = jnp.dot(xb, w3, preferred_element_type=jnp.float32)
    h = (h1 * jax.nn.sigmoid(h1)) * h3
    o = jnp.dot(h.astype(jnp.bfloat16), w2, preferred_element_type=jnp.float32)
    idx = lax.broadcasted_iota(jnp.int32, ew_ref.shape, 1)
    col = jnp.sum(jnp.where(idx == e, ew_ref[...], 0.0), axis=1, keepdims=True)
    acc_ref[...] += o * col

    @pl.when(e == NEXP - 1)
    def _():
        out_ref[...] = acc_ref[...].reshape(out_ref.shape)


def kernel(inputs, Wg, W1, W2, W3):
    B, S, D = inputs.shape
    T = B * S

    g, xb = pl.pallas_call(
        _gate_body,
        out_shape=[
            jax.ShapeDtypeStruct((T, NEXP), jnp.float32),
            jax.ShapeDtypeStruct((T, D), jnp.bfloat16),
        ],
    )(inputs, Wg)

    ew = _routing_sc(g)

    out = pl.pallas_call(
        _ffn_body,
        grid=(NEXP,),
        in_specs=[
            pl.BlockSpec((T, NEXP), lambda e: (0, 0)),
            pl.BlockSpec((T, D), lambda e: (0, 0)),
            pl.BlockSpec((1, D, INTER), lambda e: (e, 0, 0)),
            pl.BlockSpec((1, INTER, D), lambda e: (e, 0, 0)),
            pl.BlockSpec((1, D, INTER), lambda e: (e, 0, 0)),
        ],
        out_specs=pl.BlockSpec((B, S, D), lambda e: (0, 0, 0)),
        out_shape=jax.ShapeDtypeStruct((B, S, D), jnp.float32),
        scratch_shapes=[
            pltpu.VMEM((T, D), jnp.float32),
        ],
        compiler_params=pltpu.CompilerParams(
            dimension_semantics=("arbitrary",),
        ),
    )(ew, xb, W1, W2, W3)
    return out


# speedup vs baseline: 1.0068x; 1.0068x over previous
import jax
import jax.numpy as jnp
from jax import lax
from jax.experimental import pallas as pl
from jax.experimental.pallas import tpu as pltpu
from jax.experimental.pallas import tpu_sc as plsc

EMBED = 768
INTER = 1024
NEXP = 64
NTOK = 128

_INFO = plsc.get_sparse_core_info()
_NC = 1
_NS = _INFO.num_subcores
_L = _INFO.num_lanes
_NW = _NC * _NS
_TPW = NTOK // _NW
_CH = NEXP // _L


def _gate_body(x_ref, wg_ref, g_ref, xb_ref):
    x = x_ref[...].reshape(NTOK, EMBED)
    g_ref[...] = jnp.dot(x, wg_ref[...], preferred_element_type=jnp.float32)
    xb_ref[...] = x.astype(jnp.bfloat16)


def _route_body(g_hbm, ew_hbm, g_v, ew_v, bf1, bf2, bi1, bi2, sem):
    wid = lax.axis_index("s") * _NC + lax.axis_index("c")
    base = wid * _TPW
    pltpu.sync_copy(g_hbm.at[pl.ds(base, _TPW), :], g_v)
    iota = lax.iota(jnp.int32, _L)

    def _rot(x, s, buf):
        buf[pl.ds(0, _L)] = x
        buf[pl.ds(_L, _L)] = x
        return buf[pl.ds(s, _L)]

    def _token(k, _):
        v1 = g_v[k, pl.ds(0, _L)]
        i1 = iota
        v2 = jnp.full((_L,), -jnp.inf, jnp.float32)
        i2 = jnp.full((_L,), NEXP, jnp.int32)
        for c in range(1, _CH):
            g = g_v[k, pl.ds(c * _L, _L)]
            gi = iota + c * _L
            is1 = g > v1
            gt2 = g > v2
            v2 = jnp.where(is1, v1, jnp.where(gt2, g, v2))
            i2 = jnp.where(is1, i1, jnp.where(gt2, gi, i2))
            v1 = jnp.where(is1, g, v1)
            i1 = jnp.where(is1, gi, i1)
        for s in (8, 4, 2, 1):
            rv1 = _rot(v1, s, bf1)
            ri1 = _rot(i1, s, bi1)
            rv2 = _rot(v2, s, bf2)
            ri2 = _rot(i2, s, bi2)
            a_top = (v1 > rv1) | ((v1 == rv1) & (i1 < ri1))
            n1 = jnp.where(a_top, v1, rv1)
            j1 = jnp.where(a_top, i1, ri1)
            lo_v = jnp.where(a_top, rv1, v1)
            lo_i = jnp.where(a_top, ri1, i1)
            cd_v = jnp.where(a_top, v2, rv2)
            cd_i = jnp.where(a_top, i2, ri2)
            lo_top = (lo_v > cd_v) | ((lo_v == cd_v) & (lo_i < cd_i))
            v2 = jnp.where(lo_top, lo_v, cd_v)
            i2 = jnp.where(lo_top, lo_i, cd_i)
            v1, i1 = n1, j1
        w1v = 1.0 / (1.0 + jnp.exp(v2 - v1))
        w2v = 1.0 - w1v
        zero = jnp.zeros((_L,), jnp.float32)
        for c in range(_CH):
            ic = iota + c * _L
            val = (jnp.where(ic == i1, w1v, zero)
                   + jnp.where(ic == i2, w2v, zero))
            ew_v[k, pl.ds(c * _L, _L)] = val
        return 0

    lax.fori_loop(0, _TPW, _token, 0)
    pltpu.sync_copy(ew_v, ew_hbm.at[pl.ds(base, _TPW), :])


def _routing_sc(g):
    mesh = plsc.VectorSubcoreMesh(core_axis_name="c", subcore_axis_name="s",
                                  num_cores=_NC)
    return pl.kernel(
        _route_body,
        out_type=jax.ShapeDtypeStruct((NTOK, NEXP), jnp.float32),
        mesh=mesh,
        scratch_types=[
            pltpu.VMEM((_TPW, NEXP), jnp.float32),
            pltpu.VMEM((_TPW, NEXP), jnp.float32),
            pltpu.VMEM((2 * _L,), jnp.float32),
            pltpu.VMEM((2 * _L,), jnp.float32),
            pltpu.VMEM((2 * _L,), jnp.int32),
            pltpu.VMEM((2 * _L,), jnp.int32),
            pltpu.SemaphoreType.DMA,
        ],
    )(g)


def _ffn_body(ew_ref, xb_ref, w1_ref, w2_ref, w3_ref, out_ref, acc_ref):
    e = pl.program_id(0)

    @pl.when(e == 0)
    def _():
        acc_ref[...] = jnp.zeros_like(acc_ref)

    xb = xb_ref[...]
    w1 = w1_ref[0].astype(jnp.bfloat16)
    w3 = w3_ref[0].astype(jnp.bfloat16)
    w2 = w2_ref[0].astype(jnp.bfloat16)
    h1 = jnp.dot(xb, w1, preferred_element_type=jnp.float32)
    h3 = jnp.dot(xb, w3, preferred_element_type=jnp.float32)
    h = (h1 * jax.nn.sigmoid(h1)) * h3
    o = jnp.dot(h.astype(jnp.bfloat16), w2, preferred_element_type=jnp.float32)
    idx = lax.broadcasted_iota(jnp.int32, ew_ref.shape, 1)
    col = jnp.sum(jnp.where(idx == e, ew_ref[...], 0.0), axis=1, keepdims=True)
    acc_ref[...] += o * col

    @pl.when(e == NEXP - 1)
    def _():
        out_ref[...] = acc_ref[...].reshape(out_ref.shape)


def kernel(inputs, Wg, W1, W2, W3):
    B, S, D = inputs.shape
    T = B * S

    g, xb = pl.pallas_call(
        _gate_body,
        out_shape=[
            jax.ShapeDtypeStruct((T, NEXP), jnp.float32),
            jax.ShapeDtypeStruct((T, D), jnp.bfloat16),
        ],
    )(inputs, Wg)

    ew = _routing_sc(g)

    out = pl.pallas_call(
        _ffn_body,
        grid=(NEXP,),
        in_specs=[
            pl.BlockSpec((T, NEXP), lambda e: (0, 0)),
            pl.BlockSpec((T, D), lambda e: (0, 0)),
            pl.BlockSpec((1, D, INTER), lambda e: (e, 0, 0)),
            pl.BlockSpec((1, INTER, D), lambda e: (e, 0, 0)),
            pl.BlockSpec((1, D, INTER), lambda e: (e, 0, 0)),
        ],
        out_specs=pl.BlockSpec((B, S, D), lambda e: (0, 0, 0)),
        out_shape=jax.ShapeDtypeStruct((B, S, D), jnp.float32),
        scratch_shapes=[
            pltpu.VMEM((T, D), jnp.float32),
        ],
        compiler_params=pltpu.CompilerParams(
            dimension_semantics=("arbitrary",),
        ),
    )(ew, xb, W1, W2, W3)
    return out
